# baseline (speedup 1.0000x reference)
import functools

import jax
import jax.numpy as jnp
from jax import lax
from jax.experimental import pallas as pl
from jax.experimental.pallas import tpu as pltpu
from jax.experimental.pallas import tpu_sc as plsc

_NBUF = 2
_NW = 32
_PC = 8
_L = 16
_EB = 8


def _sc_gather(n, s):
    per_w = n // _NW
    q_per_w = per_w // s
    rows_e = s * _PC
    mesh = plsc.VectorSubcoreMesh(core_axis_name="c", subcore_axis_name="s")

    @functools.partial(
        pl.kernel,
        mesh=mesh,
        compiler_params=pltpu.CompilerParams(needs_layout_passes=False),
        out_type=jax.ShapeDtypeStruct((n * _PC, 128), jnp.float32),
        scratch_types=[
            pltpu.VMEM((per_w,), jnp.int32),
            pltpu.VMEM((per_w * _PC,), jnp.int32),
            [pltpu.VMEM((rows_e, 128), jnp.float32)] * _NBUF,
            [pltpu.SemaphoreType.DMA] * _NBUF,
            [pltpu.SemaphoreType.DMA] * _NBUF,
        ],
    )
    def k(idx_hbm, table_hbm, out_hbm, idx_v, idx8_v, bufs, gsems, ssems):
        nc = lax.axis_size("c")
        wid = lax.axis_index("s") * nc + lax.axis_index("c")
        q0 = wid * q_per_w
        pltpu.sync_copy(idx_hbm.at[pl.ds(wid * per_w, per_w)], idx_v)

        lanes = lax.iota(jnp.int32, _L)

        @pl.loop(0, per_w * _PC // _L)
        def expand(vv):
            n_vec = vv * _L + lanes
            g = plsc.load_gather(idx_v, [n_vec >> 3])
            idx8_v[pl.ds(vv * _L, _L)] = g * _PC + (n_vec & 7)

        def gather(e, bf):
            ebase = e * rows_e
            for off in range(0, rows_e, 128):
                sz = min(128, rows_e - off)
                pltpu.async_copy(
                    table_hbm.at[idx8_v.at[pl.ds(ebase + off, sz)]],
                    bufs[bf].at[pl.ds(off, sz)],
                    gsems[bf],
                )

        def scatter(e, bf):
            pltpu.async_copy(
                bufs[bf], out_hbm.at[pl.ds((q0 + e) * rows_e, rows_e)],
                ssems[bf],
            )

        def gwait(bf):
            pltpu.make_async_copy(
                table_hbm.at[pl.ds(0, rows_e)], bufs[bf], gsems[bf]
            ).wait()

        def swait(bf):
            pltpu.make_async_copy(
                bufs[bf], out_hbm.at[pl.ds(0, rows_e)], ssems[bf]
            ).wait()

        for bf in range(_NBUF):
            gather(bf, bf)

        @pl.loop(_NBUF, q_per_w, step=_NBUF)
        def body(e0):
            for bf in range(_NBUF):
                gwait(bf)
                scatter(e0 - _NBUF + bf, bf)
            for bf in range(_NBUF):
                swait(bf)
                gather(e0 + bf, bf)

        for bf in range(_NBUF):
            gwait(bf)
            scatter(q_per_w - _NBUF + bf, bf)
        for bf in range(_NBUF):
            swait(bf)

    return k


def _tc_layout(b, s, d):
    rows_e = s * _PC

    def body(in_ref, out_ref):
        x = in_ref[...].reshape(_EB, s, _PC, 128)
        for e in range(_EB):
            for p in range(_PC):
                w = min(128, d - p * 128)
                if w <= 0:
                    break
                out_ref[e, :, p * 128:p * 128 + w] = x[e, :, p, :w]

    return pl.pallas_call(
        body,
        grid=(b // _EB,),
        in_specs=[pl.BlockSpec((_EB * rows_e, 128), lambda i: (i, 0))],
        out_specs=pl.BlockSpec((_EB, s, d), lambda i: (i, 0, 0)),
        out_shape=jax.ShapeDtypeStruct((b, s, d), jnp.float32),
    )


def kernel(x, embedding):
    b, s = x.shape
    v, d = embedding.shape
    table8 = jnp.pad(embedding, ((0, 0), (0, _PC * 128 - d)))
    table8 = table8.reshape(v * _PC, 128)
    nchunks = 2
    bc = b // nchunks
    xi = x.reshape(nchunks, bc * s).astype(jnp.int32)
    parts = []
    for k in range(nchunks):
        staged = _sc_gather(bc * s, s)(xi[k], table8)
        parts.append(staged.reshape(bc, s, _PC * 128)[:, :, :d])
    return jnp.concatenate(parts, axis=0)


# speedup vs baseline: 1.2176x; 1.2176x over previous
import functools

import jax
import jax.numpy as jnp
from jax import lax
from jax.experimental import pallas as pl
from jax.experimental.pallas import tpu as pltpu
from jax.experimental.pallas import tpu_sc as plsc

_NBUF = 2
_NW = 32
_PC = 8
_L = 16


def _sc_gather(n, s):
    per_w = n // _NW
    q_per_w = per_w // s
    rows_e = s * _PC
    mesh = plsc.VectorSubcoreMesh(core_axis_name="c", subcore_axis_name="s")

    @functools.partial(
        pl.kernel,
        mesh=mesh,
        compiler_params=pltpu.CompilerParams(needs_layout_passes=False),
        out_type=jax.ShapeDtypeStruct((n * _PC, 128), jnp.float32),
        scratch_types=[
            pltpu.VMEM((per_w,), jnp.int32),
            pltpu.VMEM((per_w * _PC,), jnp.int32),
            [pltpu.VMEM((rows_e, 128), jnp.float32)] * _NBUF,
            [pltpu.SemaphoreType.DMA] * _NBUF,
            [pltpu.SemaphoreType.DMA] * _NBUF,
        ],
    )
    def k(idx_hbm, table_hbm, out_hbm, idx_v, idx8_v, bufs, gsems, ssems):
        nc = lax.axis_size("c")
        wid = lax.axis_index("s") * nc + lax.axis_index("c")
        q0 = wid * q_per_w
        pltpu.sync_copy(idx_hbm.at[pl.ds(wid * per_w, per_w)], idx_v)

        lanes = lax.iota(jnp.int32, _L)

        @pl.loop(0, per_w * _PC // _L)
        def expand(vv):
            n_vec = vv * _L + lanes
            g = plsc.load_gather(idx_v, [n_vec >> 3])
            idx8_v[pl.ds(vv * _L, _L)] = g * _PC + (n_vec & 7)

        def gather(e, bf):
            ebase = e * rows_e
            for off in range(0, rows_e, 128):
                sz = min(128, rows_e - off)
                pltpu.async_copy(
                    table_hbm.at[idx8_v.at[pl.ds(ebase + off, sz)]],
                    bufs[bf].at[pl.ds(off, sz)],
                    gsems[bf],
                )

        def scatter(e, bf):
            pltpu.async_copy(
                bufs[bf], out_hbm.at[pl.ds((q0 + e) * rows_e, rows_e)],
                ssems[bf],
            )

        def gwait(bf):
            pltpu.make_async_copy(
                table_hbm.at[pl.ds(0, rows_e)], bufs[bf], gsems[bf]
            ).wait()

        def swait(bf):
            pltpu.make_async_copy(
                bufs[bf], out_hbm.at[pl.ds(0, rows_e)], ssems[bf]
            ).wait()

        for bf in range(_NBUF):
            gather(bf, bf)

        @pl.loop(_NBUF, q_per_w, step=_NBUF)
        def body(e0):
            for bf in range(_NBUF):
                gwait(bf)
                scatter(e0 - _NBUF + bf, bf)
            for bf in range(_NBUF):
                swait(bf)
                gather(e0 + bf, bf)

        for bf in range(_NBUF):
            gwait(bf)
            scatter(q_per_w - _NBUF + bf, bf)
        for bf in range(_NBUF):
            swait(bf)

    return k


def kernel(x, embedding):
    b, s = x.shape
    v, d = embedding.shape
    table8 = jnp.pad(embedding, ((0, 0), (0, _PC * 128 - d)))
    table8 = table8.reshape(v * _PC, 128)
    staged = _sc_gather(b * s, s)(x.reshape(b * s).astype(jnp.int32), table8)
    return staged.reshape(b, s, _PC * 128)[:, :, :d]


# speedup vs baseline: 1.2296x; 1.0098x over previous
import functools

import jax
import jax.numpy as jnp
from jax import lax
from jax.experimental import pallas as pl
from jax.experimental.pallas import tpu as pltpu
from jax.experimental.pallas import tpu_sc as plsc

_NBUF = 4
_NW = 32
_PC = 8
_L = 16
_HALF = 2


def _sc_gather(n, s):
    per_w = n // _NW
    q_per_w = per_w // s
    rows_e = s * _PC
    ch = rows_e // _HALF
    n_ch = q_per_w * _HALF
    mesh = plsc.VectorSubcoreMesh(core_axis_name="c", subcore_axis_name="s")

    @functools.partial(
        pl.kernel,
        mesh=mesh,
        compiler_params=pltpu.CompilerParams(needs_layout_passes=False),
        out_type=jax.ShapeDtypeStruct((n * _PC, 128), jnp.float32),
        scratch_types=[
            pltpu.VMEM((per_w,), jnp.int32),
            pltpu.VMEM((per_w * _PC,), jnp.int32),
            [pltpu.VMEM((ch, 128), jnp.float32)] * _NBUF,
            [pltpu.SemaphoreType.DMA] * _NBUF,
            [pltpu.SemaphoreType.DMA] * _NBUF,
        ],
    )
    def k(idx_hbm, table_hbm, out_hbm, idx_v, idx8_v, bufs, gsems, ssems):
        nc = lax.axis_size("c")
        wid = lax.axis_index("s") * nc + lax.axis_index("c")
        q0 = wid * q_per_w
        pltpu.sync_copy(idx_hbm.at[pl.ds(wid * per_w, per_w)], idx_v)

        lanes = lax.iota(jnp.int32, _L)

        @pl.loop(0, per_w * _PC // _L)
        def expand(vv):
            n_vec = vv * _L + lanes
            g = plsc.load_gather(idx_v, [n_vec >> 3])
            idx8_v[pl.ds(vv * _L, _L)] = g * _PC + (n_vec & 7)

        def gather(c, bf):
            cbase = c * ch
            for off in range(0, ch, 128):
                sz = min(128, ch - off)
                pltpu.async_copy(
                    table_hbm.at[idx8_v.at[pl.ds(cbase + off, sz)]],
                    bufs[bf].at[pl.ds(off, sz)],
                    gsems[bf],
                )

        def scatter(c, bf):
            pltpu.async_copy(
                bufs[bf],
                out_hbm.at[pl.ds(wid * per_w * _PC + c * ch, ch)],
                ssems[bf],
            )

        def gwait(bf):
            pltpu.make_async_copy(
                table_hbm.at[pl.ds(0, ch)], bufs[bf], gsems[bf]
            ).wait()

        def swait(bf):
            pltpu.make_async_copy(
                bufs[bf], out_hbm.at[pl.ds(0, ch)], ssems[bf]
            ).wait()

        for bf in range(_NBUF):
            gather(bf, bf)

        @pl.loop(_NBUF, n_ch, step=_NBUF)
        def body(c0):
            for bf in range(_NBUF):
                gwait(bf)
                scatter(c0 - _NBUF + bf, bf)
            for bf in range(_NBUF):
                swait(bf)
                gather(c0 + bf, bf)

        for bf in range(_NBUF):
            gwait(bf)
            scatter(n_ch - _NBUF + bf, bf)
        for bf in range(_NBUF):
            swait(bf)

    return k


def kernel(x, embedding):
    b, s = x.shape
    v, d = embedding.shape
    table8 = jnp.pad(embedding, ((0, 0), (0, _PC * 128 - d)))
    table8 = table8.reshape(v * _PC, 128)
    staged = _sc_gather(b * s, s)(x.reshape(b * s).astype(jnp.int32), table8)
    return staged.reshape(b, s, _PC * 128)[:, :, :d]
